# baseline (speedup 1.0000x reference)
import jax
import jax.numpy as jnp
from jax.experimental import pallas as pl
from jax.experimental.pallas import tpu as pltpu

NODE_NUM = 100
HIDDEN_SIZE = 4096
NCHUNK = 8
CHUNK_COLS = HIDDEN_SIZE // NCHUNK
DEPTH = 2


def _in_copy(in_hbm, buf, insem, c):
    return pltpu.make_async_copy(
        in_hbm.at[:, pl.ds(c * CHUNK_COLS, CHUNK_COLS)],
        buf.at[:, pl.ds(c * CHUNK_COLS, CHUNK_COLS)],
        insem.at[c],
    )


def _out_copy(buf, out_hbm, outsem, c):
    return pltpu.make_async_copy(
        buf.at[:, pl.ds(c * CHUNK_COLS, CHUNK_COLS)],
        out_hbm.at[:, pl.ds(c * CHUNK_COLS, CHUNK_COLS)],
        outsem.at[c],
    )


def _dma_body(in_hbm, out_hbm, buf, insem, outsem):
    for c in range(DEPTH):
        _in_copy(in_hbm, buf, insem, c).start()
    for c in range(NCHUNK):
        _in_copy(in_hbm, buf, insem, c).wait()
        _out_copy(buf, out_hbm, outsem, c).start()
        if c + DEPTH < NCHUNK:
            _in_copy(in_hbm, buf, insem, c + DEPTH).start()
    for c in range(NCHUNK):
        _out_copy(buf, out_hbm, outsem, c).wait()


def kernel(node_table):
    return pl.pallas_call(
        _dma_body,
        out_shape=jax.ShapeDtypeStruct((NODE_NUM, HIDDEN_SIZE), jnp.float32),
        in_specs=[pl.BlockSpec(memory_space=pl.ANY)],
        out_specs=pl.BlockSpec(memory_space=pl.ANY),
        scratch_shapes=[
            pltpu.VMEM((NODE_NUM, HIDDEN_SIZE), jnp.float32),
            pltpu.SemaphoreType.DMA((NCHUNK,)),
            pltpu.SemaphoreType.DMA((NCHUNK,)),
        ],
    )(node_table)


# speedup vs baseline: 1.2232x; 1.2232x over previous
import jax
import jax.numpy as jnp
from jax.experimental import pallas as pl
from jax.experimental.pallas import tpu as pltpu

NODE_NUM = 100
HIDDEN_SIZE = 4096
ROW_CHUNKS = tuple((8 * i, 8) for i in range(12)) + ((96, 4),)
NCHUNK = len(ROW_CHUNKS)
DEPTH = 5


def _in_copy(in_hbm, buf, insem, c):
    off, sz = ROW_CHUNKS[c]
    return pltpu.make_async_copy(
        in_hbm.at[pl.ds(off, sz), :],
        buf.at[pl.ds(off, sz), :],
        insem.at[c],
    )


def _out_copy(buf, out_hbm, outsem, c):
    off, sz = ROW_CHUNKS[c]
    return pltpu.make_async_copy(
        buf.at[pl.ds(off, sz), :],
        out_hbm.at[pl.ds(off, sz), :],
        outsem.at[c],
    )


def _dma_body(in_hbm, out_hbm, buf, insem, outsem):
    for c in range(DEPTH):
        _in_copy(in_hbm, buf, insem, c).start()
    for c in range(NCHUNK):
        _in_copy(in_hbm, buf, insem, c).wait()
        _out_copy(buf, out_hbm, outsem, c).start()
        if c + DEPTH < NCHUNK:
            _in_copy(in_hbm, buf, insem, c + DEPTH).start()
    for c in range(NCHUNK):
        _out_copy(buf, out_hbm, outsem, c).wait()


def kernel(node_table):
    return pl.pallas_call(
        _dma_body,
        out_shape=jax.ShapeDtypeStruct((NODE_NUM, HIDDEN_SIZE), jnp.float32),
        in_specs=[pl.BlockSpec(memory_space=pl.ANY)],
        out_specs=pl.BlockSpec(memory_space=pl.ANY),
        scratch_shapes=[
            pltpu.VMEM((NODE_NUM, HIDDEN_SIZE), jnp.float32),
            pltpu.SemaphoreType.DMA((NCHUNK,)),
            pltpu.SemaphoreType.DMA((NCHUNK,)),
        ],
    )(node_table)


# speedup vs baseline: 2.1923x; 1.7923x over previous
import jax
import jax.numpy as jnp
from jax.experimental import pallas as pl
from jax.experimental.pallas import tpu as pltpu

NODE_NUM = 100
HIDDEN_SIZE = 4096
NCHUNK = 4
CHUNK_COLS = HIDDEN_SIZE // NCHUNK


def _dma_body(in_hbm, out_hbm, buf, insem, outsem):
    for c in range(NCHUNK):
        pltpu.make_async_copy(
            in_hbm.at[:, pl.ds(c * CHUNK_COLS, CHUNK_COLS)],
            buf.at[:, pl.ds(c * CHUNK_COLS, CHUNK_COLS)],
            insem.at[c],
        ).start()
    for c in range(NCHUNK):
        pltpu.make_async_copy(
            in_hbm.at[:, pl.ds(c * CHUNK_COLS, CHUNK_COLS)],
            buf.at[:, pl.ds(c * CHUNK_COLS, CHUNK_COLS)],
            insem.at[c],
        ).wait()
        pltpu.make_async_copy(
            buf.at[:, pl.ds(c * CHUNK_COLS, CHUNK_COLS)],
            out_hbm.at[:, pl.ds(c * CHUNK_COLS, CHUNK_COLS)],
            outsem.at[c],
        ).start()
    for c in range(NCHUNK):
        pltpu.make_async_copy(
            buf.at[:, pl.ds(c * CHUNK_COLS, CHUNK_COLS)],
            out_hbm.at[:, pl.ds(c * CHUNK_COLS, CHUNK_COLS)],
            outsem.at[c],
        ).wait()


def kernel(node_table):
    return pl.pallas_call(
        _dma_body,
        out_shape=jax.ShapeDtypeStruct((NODE_NUM, HIDDEN_SIZE), jnp.float32),
        in_specs=[pl.BlockSpec(memory_space=pl.ANY)],
        out_specs=pl.BlockSpec(memory_space=pl.ANY),
        scratch_shapes=[
            pltpu.VMEM((NODE_NUM, HIDDEN_SIZE), jnp.float32),
            pltpu.SemaphoreType.DMA((NCHUNK,)),
            pltpu.SemaphoreType.DMA((NCHUNK,)),
        ],
    )(node_table)
